# Initial kernel scaffold; baseline (speedup 1.0000x reference)
import jax
import jax.numpy as jnp
from jax.experimental import pallas as pl


def kernel(support_embeddings, query_embeddings, classification_results):
    raise NotImplementedError("write your pallas kernel here")


# speedup vs baseline: 3.0053x; 3.0053x over previous
import functools

import jax
import jax.numpy as jnp
from jax import lax
from jax.experimental import pallas as pl
from jax.experimental.pallas import tpu as pltpu
from jax.experimental.pallas import tpu_sc as plsc

W = 16
KSH = 8
QSH = 32
TOPK = 32
D = 768
S = W * KSH
Q = W * QSH

CF = 64
NCF = Q // CF
CT = 64
NCT = Q // CT

_NC = 2
_NS = 16
_NW = _NC * _NS
_B = S * TOPK
_BPW = _B // _NW


def _dense_body(sup_ref, q_ref, cls_ref, idx_ref, acc_ref, work_ref, supt_ref):
    supt_ref[...] = sup_ref[...].T
    supt = supt_ref[...]
    sup_n = jnp.sum(supt * supt, axis=0, keepdims=True)
    rep = (lax.broadcasted_iota(jnp.int32, (W, S), 1) // KSH
           == lax.broadcasted_iota(jnp.int32, (W, S), 0)).astype(jnp.float32)

    for c in range(NCF):
        rows = pl.ds(c * CF, CF)
        qc = q_ref[rows, :]
        dot = jnp.dot(qc, supt, preferred_element_type=jnp.float32,
                      precision=lax.Precision.HIGHEST)
        q_n = jnp.sum(qc * qc, axis=1, keepdims=True)
        logits = 2.0 * dot - sup_n - q_n

        m = jnp.max(logits, axis=1, keepdims=True)
        e = jnp.exp(logits - m)
        p = e / jnp.sum(e, axis=1, keepdims=True)
        ent_sim = jnp.maximum(-jnp.sum(p * jnp.log(p + 0.001), axis=1,
                                       keepdims=True), 0.0)

        cc = cls_ref[rows, :]
        cm = jnp.max(cc, axis=1, keepdims=True)
        ce = jnp.exp(cc - cm)
        cp = ce / jnp.sum(ce, axis=1, keepdims=True)
        exp_p = jnp.dot(cp, rep, preferred_element_type=jnp.float32,
                        precision=lax.Precision.HIGHEST)
        ent_exp = jnp.maximum(-jnp.sum(exp_p * jnp.log(exp_p + 0.001),
                                       axis=1, keepdims=True), 0.0)

        work_ref[rows, :] = (exp_p / (1.0 + ent_exp)
                             + p / (1.0 + ent_sim))

    iota_c = lax.broadcasted_iota(jnp.int32, (CT, S), 0)
    way_lo = (lax.broadcasted_iota(jnp.int32, (1, S), 1) // KSH) * QSH
    big = jnp.int32(1 << 30)
    neg_inf = jnp.float32(-jnp.inf)

    def step(k, carry):
        prev_idx, correct = carry
        mx = jnp.full((1, S), neg_inf, jnp.float32)
        for c in range(NCT):
            rows = pl.ds(c * CT, CT)
            wc = work_ref[rows, :]
            wc = jnp.where(iota_c + (c * CT) == prev_idx, neg_inf, wc)
            work_ref[rows, :] = wc
            mx = jnp.maximum(mx, jnp.max(wc, axis=0, keepdims=True))
        idx = jnp.full((1, S), big, jnp.int32)
        for c in range(NCT):
            wc = work_ref[pl.ds(c * CT, CT), :]
            cidx = jnp.min(jnp.where(wc == mx, iota_c + (c * CT), big),
                           axis=0, keepdims=True)
            idx = jnp.minimum(idx, cidx)
        idx_ref[pl.ds(k, 1), :] = idx
        inside = jnp.logical_and(idx >= way_lo, idx <= way_lo + (QSH - 1))
        return idx, correct + inside.astype(jnp.float32)

    _, correct = lax.fori_loop(
        0, TOPK, step,
        (jnp.full((1, S), jnp.int32(-1)), jnp.zeros((1, S), jnp.float32)))
    acc_ref[...] = (jnp.sum(correct) / jnp.float32(S * TOPK)).reshape(1, 1)


_dense_call = pl.pallas_call(
    _dense_body,
    out_shape=(
        jax.ShapeDtypeStruct((TOPK, S), jnp.int32),
        jax.ShapeDtypeStruct((1, 1), jnp.float32),
    ),
    in_specs=[
        pl.BlockSpec(memory_space=pltpu.VMEM),
        pl.BlockSpec(memory_space=pltpu.VMEM),
        pl.BlockSpec(memory_space=pltpu.VMEM),
    ],
    out_specs=(
        pl.BlockSpec(memory_space=pltpu.VMEM),
        pl.BlockSpec(memory_space=pltpu.VMEM),
    ),
    scratch_shapes=[
        pltpu.VMEM((Q, S), jnp.float32),
        pltpu.VMEM((D, S), jnp.float32),
    ],
)


def _sc_gather_body(table_hbm, idx_hbm, out_hbm, idx_v, rows_v, sem):
    wid = lax.axis_index("s") * _NC + lax.axis_index("c")
    base = wid * _BPW
    pltpu.sync_copy(idx_hbm.at[pl.ds(base, _BPW)], idx_v)
    pltpu.async_copy(table_hbm.at[idx_v], rows_v, sem).wait()
    pltpu.sync_copy(rows_v, out_hbm.at[pl.ds(base, _BPW)])


@functools.cache
def _sc_gather():
    return pl.kernel(
        _sc_gather_body,
        out_type=jax.ShapeDtypeStruct((_B, D), jnp.float32),
        mesh=plsc.VectorSubcoreMesh(core_axis_name="c", subcore_axis_name="s"),
        scratch_types=[
            pltpu.VMEM((_BPW,), jnp.int32),
            pltpu.VMEM((_BPW, D), jnp.float32),
            pltpu.SemaphoreType.DMA,
        ],
    )


def kernel(support_embeddings, query_embeddings, classification_results):
    idx_km, acc = _dense_call(support_embeddings, query_embeddings,
                              classification_results)
    flat_idx = idx_km.T.reshape(_B)
    sampled = _sc_gather()(query_embeddings, flat_idx)
    return sampled.reshape(W, KSH * TOPK, D), acc[0, 0]
